# baseline (speedup 1.0000x reference)
import functools

import jax
import jax.numpy as jnp
from jax import lax
from jax.experimental import pallas as pl
from jax.experimental.pallas import tpu as pltpu
from jax.experimental.pallas import tpu_sc as plsc

NN = 10000
DD = 128
FF = 16
NC = 2
NS = 16
LL = 16
NW = NC * NS
NP = 10240
RPT = NP // NS
CH = 128

_mesh = plsc.VectorSubcoreMesh(core_axis_name="c", subcore_axis_name="s")


def _make_deg_kernel(K):
    @functools.partial(
        pl.kernel,
        out_type=jax.ShapeDtypeStruct((NC, NP), jnp.float32),
        mesh=_mesh,
        scratch_types=(
            pltpu.VMEM((K, CH), jnp.int32),
            pltpu.VMEM((CH,), jnp.float32),
            pltpu.VMEM((RPT,), jnp.float32),
            pltpu.VMEM_SHARED((NP,), jnp.float32),
        ),
    )
    def deg_kernel(dst_hbm, out_hbm, dst_v, ones_v, zero_v, deg_sh):
        c = lax.axis_index("c")
        s = lax.axis_index("s")
        wid = s * NC + c
        for i in range(CH // LL):
            ones_v[pl.ds(i * LL, LL)] = jnp.full((LL,), 1.0, jnp.float32)

        def zbody(i, carry):
            zero_v[pl.ds(i * LL, LL)] = jnp.zeros((LL,), jnp.float32)
            return carry

        lax.fori_loop(0, RPT // LL, zbody, 0)
        pltpu.sync_copy(zero_v, deg_sh.at[pl.ds(s * RPT, RPT)])
        pltpu.sync_copy(dst_hbm.at[wid], dst_v)
        plsc.subcore_barrier()

        def body(j, carry):
            pltpu.sync_copy(ones_v, deg_sh.at[dst_v.at[j]], add=True)
            return carry

        lax.fori_loop(0, K, body, 0)
        plsc.subcore_barrier()
        pltpu.sync_copy(deg_sh.at[pl.ds(s * RPT, RPT)],
                        out_hbm.at[c, pl.ds(s * RPT, RPT)])

    return deg_kernel


def _make_prop_kernel(K):
    @functools.partial(
        pl.kernel,
        out_type=jax.ShapeDtypeStruct((NC, NP, FF), jnp.float32),
        mesh=_mesh,
        scratch_types=(
            pltpu.VMEM((K, CH), jnp.int32),
            pltpu.VMEM((K, CH), jnp.int32),
            pltpu.VMEM((CH, FF), jnp.float32),
            pltpu.VMEM((CH, FF), jnp.float32),
            pltpu.VMEM((RPT, FF), jnp.float32),
            pltpu.VMEM_SHARED((NP, FF), jnp.float32),
            pltpu.SemaphoreType.DMA,
            pltpu.SemaphoreType.DMA,
            pltpu.SemaphoreType.DMA,
            pltpu.SemaphoreType.DMA,
        ),
        compiler_params=pltpu.CompilerParams(use_tc_tiling_on_sc=False),
    )
    def prop_kernel(y_hbm, src_hbm, dst_hbm, out_hbm,
                    src_v, dst_v, rows0_v, rows1_v, zero_v, acc_sh,
                    sem0, sem1, ssem0, ssem1):
        c = lax.axis_index("c")
        s = lax.axis_index("s")
        wid = s * NC + c

        def zbody(i, carry):
            zero_v[i, :] = jnp.zeros((FF,), jnp.float32)
            return carry

        lax.fori_loop(0, RPT, zbody, 0)
        pltpu.sync_copy(zero_v, acc_sh.at[pl.ds(s * RPT, RPT)])
        pltpu.sync_copy(src_hbm.at[wid], src_v)
        pltpu.sync_copy(dst_hbm.at[wid], dst_v)
        plsc.subcore_barrier()

        pltpu.async_copy(y_hbm.at[src_v.at[0]], rows0_v, sem0)

        def body(o, carry):
            j0 = 2 * o
            j1 = j0 + 1
            pltpu.make_async_copy(y_hbm.at[src_v.at[j0]], rows0_v, sem0).wait()
            pltpu.async_copy(y_hbm.at[src_v.at[j1]], rows1_v, sem1)
            pltpu.async_copy(rows0_v, acc_sh.at[dst_v.at[j0]], ssem0, add=True)
            pltpu.make_async_copy(y_hbm.at[src_v.at[j1]], rows1_v, sem1).wait()
            pltpu.async_copy(rows1_v, acc_sh.at[dst_v.at[j1]], ssem1, add=True)
            pltpu.make_async_copy(rows0_v, acc_sh.at[dst_v.at[j0]], ssem0).wait()

            @pl.when(o + 1 < K // 2)
            def _():
                pltpu.async_copy(y_hbm.at[src_v.at[j0 + 2]], rows0_v, sem0)

            pltpu.make_async_copy(rows1_v, acc_sh.at[dst_v.at[j1]], ssem1).wait()
            return carry

        lax.fori_loop(0, K // 2, body, 0)
        plsc.subcore_barrier()
        pltpu.sync_copy(acc_sh.at[pl.ds(s * RPT, RPT)],
                        out_hbm.at[c, pl.ds(s * RPT, RPT)])

    return prop_kernel


BR2 = 1024
BR6 = 400
BRB = 200


def _mm1_body(x_ref, w1_ref, xw_ref):
    xw_ref[...] = jnp.dot(x_ref[...], w1_ref[...],
                          preferred_element_type=jnp.float32)


_mm1_call = pl.pallas_call(
    _mm1_body,
    grid=(NP // BR2,),
    in_specs=[
        pl.BlockSpec((BR2, DD), lambda i: (i, 0)),
        pl.BlockSpec((DD, FF), lambda i: (0, 0)),
    ],
    out_specs=pl.BlockSpec((BR2, FF), lambda i: (i, 0)),
    out_shape=jax.ShapeDtypeStruct((NP, FF), jnp.float32),
)


def _scale1_body(deg_ref, xw_ref, y1_ref):
    deg = deg_ref[:, 0:1] + deg_ref[:, 1:2] + 1.0
    dinv = lax.rsqrt(deg)
    y1_ref[...] = xw_ref[...] * dinv


_scale1_call = pl.pallas_call(
    _scale1_body,
    grid=(NP // BR2,),
    in_specs=[
        pl.BlockSpec((BR2, NC), lambda i: (i, 0)),
        pl.BlockSpec((BR2, FF), lambda i: (i, 0)),
    ],
    out_specs=pl.BlockSpec((BR2, FF), lambda i: (i, 0)),
    out_shape=jax.ShapeDtypeStruct((NP, FF), jnp.float32),
)


def _l2_body(acc_ref, deg_ref, xw_ref, b1_ref, w2_ref, y2_ref, hw2_ref):
    deg = deg_ref[:, 0:1] + deg_ref[:, 1:2] + 1.0
    dinv = lax.rsqrt(deg)
    accsum = acc_ref[0] + acc_ref[1]
    h = jnp.maximum(dinv * accsum + (dinv * dinv) * xw_ref[...] + b1_ref[...], 0.0)
    hw2 = jnp.dot(h, w2_ref[...], preferred_element_type=jnp.float32)
    hw2_ref[...] = hw2
    y2_ref[...] = hw2 * dinv


_l2_call = pl.pallas_call(
    _l2_body,
    grid=(NP // BR2,),
    in_specs=[
        pl.BlockSpec((NC, BR2, FF), lambda i: (0, i, 0)),
        pl.BlockSpec((BR2, NC), lambda i: (i, 0)),
        pl.BlockSpec((BR2, FF), lambda i: (i, 0)),
        pl.BlockSpec((1, FF), lambda i: (0, 0)),
        pl.BlockSpec((FF, FF), lambda i: (0, 0)),
    ],
    out_specs=[
        pl.BlockSpec((BR2, FF), lambda i: (i, 0)),
        pl.BlockSpec((BR2, FF), lambda i: (i, 0)),
    ],
    out_shape=[
        jax.ShapeDtypeStruct((NP, FF), jnp.float32),
        jax.ShapeDtypeStruct((NP, FF), jnp.float32),
    ],
)


def _zlogp_body(acc_ref, deg_ref, hw2_ref, b2_ref, z_ref, logp_ref):
    deg = deg_ref[:, 0:1] + deg_ref[:, 1:2] + 1.0
    dinv = lax.rsqrt(deg)
    accsum = acc_ref[0] + acc_ref[1]
    z = dinv * accsum + (dinv * dinv) * hw2_ref[...] + b2_ref[...]
    z_ref[...] = z
    zmax = jnp.max(z, axis=1, keepdims=True)
    ez = jnp.exp(z - zmax)
    lse = jnp.log(jnp.sum(ez, axis=1, keepdims=True)) + zmax
    logp_ref[...] = z - lse


_zlogp_call = pl.pallas_call(
    _zlogp_body,
    grid=(NN // BR6,),
    in_specs=[
        pl.BlockSpec((NC, BR6, FF), lambda i: (0, i, 0)),
        pl.BlockSpec((BR6, NC), lambda i: (i, 0)),
        pl.BlockSpec((BR6, FF), lambda i: (i, 0)),
        pl.BlockSpec((1, FF), lambda i: (0, 0)),
    ],
    out_specs=[
        pl.BlockSpec((BR6, FF), lambda i: (i, 0)),
        pl.BlockSpec((BR6, FF), lambda i: (i, 0)),
    ],
    out_shape=[
        jax.ShapeDtypeStruct((NN, FF), jnp.float32),
        jax.ShapeDtypeStruct((NN, FF), jnp.float32),
    ],
)


def _decoder_body(zr_ref, za_ref, out_ref):
    out_ref[...] = lax.dot_general(
        zr_ref[...], za_ref[...],
        dimension_numbers=(((1,), (1,)), ((), ())),
        preferred_element_type=jnp.float32,
    )


_decoder_call = pl.pallas_call(
    _decoder_body,
    grid=(NN // BRB,),
    in_specs=[
        pl.BlockSpec((BRB, FF), lambda i: (i, 0)),
        pl.BlockSpec((NN, FF), lambda i: (0, 0)),
    ],
    out_specs=pl.BlockSpec((BRB, NN), lambda i: (i, 0)),
    out_shape=jax.ShapeDtypeStruct((NN, NN), jnp.float32),
)


def kernel(x, edge_index, W1, b1, W2, b2):
    E = edge_index.shape[1]
    epw = -(-E // NW)
    K = -(-epw // CH)
    K += K % 2
    EP = NW * K * CH
    pad = EP - E

    src = edge_index[0]
    dst = edge_index[1]
    padidx = NN + (jnp.arange(pad, dtype=jnp.int32) % (NP - NN))
    src_p = jnp.concatenate([src, padidx]).reshape(NW, K, CH)
    dst_p = jnp.concatenate([dst, padidx]).reshape(NW, K, CH)
    x_pad = jnp.pad(x, ((0, NP - NN), (0, 0)))

    deg_call = _make_deg_kernel(K)
    prop_call = _make_prop_kernel(K)

    degp = deg_call(dst_p)
    degp_t = degp.T
    xw1 = _mm1_call(x_pad, W1)
    y1 = _scale1_call(degp_t, xw1)
    acc1 = prop_call(y1, src_p, dst_p)
    y2, hw2 = _l2_call(acc1, degp_t, xw1, b1.reshape(1, FF), W2)
    acc2 = prop_call(y2, src_p, dst_p)
    z, logp = _zlogp_call(acc2, degp_t, hw2, b2.reshape(1, FF))
    x_product = _decoder_call(z, z)
    return (logp, x_product)


# speedup vs baseline: 1.0018x; 1.0018x over previous
import functools

import jax
import jax.numpy as jnp
from jax import lax
from jax.experimental import pallas as pl
from jax.experimental.pallas import tpu as pltpu
from jax.experimental.pallas import tpu_sc as plsc

NN = 10000
DD = 128
FF = 16
NC = 2
NS = 16
LL = 16
NW = NC * NS
NP = 10240
RPT = NP // NS
CH = 128

_mesh = plsc.VectorSubcoreMesh(core_axis_name="c", subcore_axis_name="s")


def _make_deg_kernel(K):
    @functools.partial(
        pl.kernel,
        out_type=jax.ShapeDtypeStruct((NC, NP), jnp.float32),
        mesh=_mesh,
        scratch_types=(
            pltpu.VMEM((K, CH), jnp.int32),
            pltpu.VMEM((CH,), jnp.float32),
            pltpu.VMEM((RPT,), jnp.float32),
            pltpu.VMEM_SHARED((NP,), jnp.float32),
        ),
    )
    def deg_kernel(dst_hbm, out_hbm, dst_v, ones_v, zero_v, deg_sh):
        c = lax.axis_index("c")
        s = lax.axis_index("s")
        wid = s * NC + c
        for i in range(CH // LL):
            ones_v[pl.ds(i * LL, LL)] = jnp.full((LL,), 1.0, jnp.float32)

        def zbody(i, carry):
            zero_v[pl.ds(i * LL, LL)] = jnp.zeros((LL,), jnp.float32)
            return carry

        lax.fori_loop(0, RPT // LL, zbody, 0)
        pltpu.sync_copy(zero_v, deg_sh.at[pl.ds(s * RPT, RPT)])
        pltpu.sync_copy(dst_hbm.at[wid], dst_v)
        plsc.subcore_barrier()

        def body(j, carry):
            pltpu.sync_copy(ones_v, deg_sh.at[dst_v.at[j]], add=True)
            return carry

        lax.fori_loop(0, K, body, 0)
        plsc.subcore_barrier()
        pltpu.sync_copy(deg_sh.at[pl.ds(s * RPT, RPT)],
                        out_hbm.at[c, pl.ds(s * RPT, RPT)])

    return deg_kernel


def _make_prop_kernel(K):
    @functools.partial(
        pl.kernel,
        out_type=jax.ShapeDtypeStruct((NC, NP, FF), jnp.float32),
        mesh=_mesh,
        scratch_types=(
            pltpu.VMEM((K, CH), jnp.int32),
            pltpu.VMEM((K, CH), jnp.int32),
            pltpu.VMEM((CH, FF), jnp.float32),
            pltpu.VMEM((CH, FF), jnp.float32),
            pltpu.VMEM((RPT, FF), jnp.float32),
            pltpu.VMEM_SHARED((NP, FF), jnp.float32),
            pltpu.SemaphoreType.DMA,
            pltpu.SemaphoreType.DMA,
        ),
        compiler_params=pltpu.CompilerParams(use_tc_tiling_on_sc=False),
    )
    def prop_kernel(y_hbm, src_hbm, dst_hbm, out_hbm,
                    src_v, dst_v, rows0_v, rows1_v, zero_v, acc_sh,
                    sem0, sem1):
        c = lax.axis_index("c")
        s = lax.axis_index("s")
        wid = s * NC + c

        def zbody(i, carry):
            zero_v[i, :] = jnp.zeros((FF,), jnp.float32)
            return carry

        lax.fori_loop(0, RPT, zbody, 0)
        pltpu.sync_copy(zero_v, acc_sh.at[pl.ds(s * RPT, RPT)])
        pltpu.sync_copy(src_hbm.at[wid], src_v)
        pltpu.sync_copy(dst_hbm.at[wid], dst_v)
        plsc.subcore_barrier()

        pltpu.async_copy(y_hbm.at[src_v.at[0]], rows0_v, sem0)

        def body(o, carry):
            j0 = 2 * o
            j1 = j0 + 1
            pltpu.make_async_copy(y_hbm.at[src_v.at[j0]], rows0_v, sem0).wait()
            pltpu.async_copy(y_hbm.at[src_v.at[j1]], rows1_v, sem1)
            pltpu.sync_copy(rows0_v, acc_sh.at[dst_v.at[j0]], add=True)
            pltpu.make_async_copy(y_hbm.at[src_v.at[j1]], rows1_v, sem1).wait()

            @pl.when(o + 1 < K // 2)
            def _():
                pltpu.async_copy(y_hbm.at[src_v.at[j0 + 2]], rows0_v, sem0)

            pltpu.sync_copy(rows1_v, acc_sh.at[dst_v.at[j1]], add=True)
            return carry

        lax.fori_loop(0, K // 2, body, 0)
        plsc.subcore_barrier()
        pltpu.sync_copy(acc_sh.at[pl.ds(s * RPT, RPT)],
                        out_hbm.at[c, pl.ds(s * RPT, RPT)])

    return prop_kernel


BR2 = 1024
BR6 = 400
BRB = 200


def _mm1_body(x_ref, w1_ref, xw_ref):
    xw_ref[...] = jnp.dot(x_ref[...], w1_ref[...],
                          preferred_element_type=jnp.float32)


_mm1_call = pl.pallas_call(
    _mm1_body,
    grid=(NP // BR2,),
    in_specs=[
        pl.BlockSpec((BR2, DD), lambda i: (i, 0)),
        pl.BlockSpec((DD, FF), lambda i: (0, 0)),
    ],
    out_specs=pl.BlockSpec((BR2, FF), lambda i: (i, 0)),
    out_shape=jax.ShapeDtypeStruct((NP, FF), jnp.float32),
)


def _scale1_body(deg_ref, xw_ref, y1_ref):
    deg = deg_ref[:, 0:1] + deg_ref[:, 1:2] + 1.0
    dinv = lax.rsqrt(deg)
    y1_ref[...] = xw_ref[...] * dinv


_scale1_call = pl.pallas_call(
    _scale1_body,
    grid=(NP // BR2,),
    in_specs=[
        pl.BlockSpec((BR2, NC), lambda i: (i, 0)),
        pl.BlockSpec((BR2, FF), lambda i: (i, 0)),
    ],
    out_specs=pl.BlockSpec((BR2, FF), lambda i: (i, 0)),
    out_shape=jax.ShapeDtypeStruct((NP, FF), jnp.float32),
)


def _l2_body(acc_ref, deg_ref, xw_ref, b1_ref, w2_ref, y2_ref, hw2_ref):
    deg = deg_ref[:, 0:1] + deg_ref[:, 1:2] + 1.0
    dinv = lax.rsqrt(deg)
    accsum = acc_ref[0] + acc_ref[1]
    h = jnp.maximum(dinv * accsum + (dinv * dinv) * xw_ref[...] + b1_ref[...], 0.0)
    hw2 = jnp.dot(h, w2_ref[...], preferred_element_type=jnp.float32)
    hw2_ref[...] = hw2
    y2_ref[...] = hw2 * dinv


_l2_call = pl.pallas_call(
    _l2_body,
    grid=(NP // BR2,),
    in_specs=[
        pl.BlockSpec((NC, BR2, FF), lambda i: (0, i, 0)),
        pl.BlockSpec((BR2, NC), lambda i: (i, 0)),
        pl.BlockSpec((BR2, FF), lambda i: (i, 0)),
        pl.BlockSpec((1, FF), lambda i: (0, 0)),
        pl.BlockSpec((FF, FF), lambda i: (0, 0)),
    ],
    out_specs=[
        pl.BlockSpec((BR2, FF), lambda i: (i, 0)),
        pl.BlockSpec((BR2, FF), lambda i: (i, 0)),
    ],
    out_shape=[
        jax.ShapeDtypeStruct((NP, FF), jnp.float32),
        jax.ShapeDtypeStruct((NP, FF), jnp.float32),
    ],
)


def _zlogp_body(acc_ref, deg_ref, hw2_ref, b2_ref, z_ref, logp_ref):
    deg = deg_ref[:, 0:1] + deg_ref[:, 1:2] + 1.0
    dinv = lax.rsqrt(deg)
    accsum = acc_ref[0] + acc_ref[1]
    z = dinv * accsum + (dinv * dinv) * hw2_ref[...] + b2_ref[...]
    z_ref[...] = z
    zmax = jnp.max(z, axis=1, keepdims=True)
    ez = jnp.exp(z - zmax)
    lse = jnp.log(jnp.sum(ez, axis=1, keepdims=True)) + zmax
    logp_ref[...] = z - lse


_zlogp_call = pl.pallas_call(
    _zlogp_body,
    grid=(NN // BR6,),
    in_specs=[
        pl.BlockSpec((NC, BR6, FF), lambda i: (0, i, 0)),
        pl.BlockSpec((BR6, NC), lambda i: (i, 0)),
        pl.BlockSpec((BR6, FF), lambda i: (i, 0)),
        pl.BlockSpec((1, FF), lambda i: (0, 0)),
    ],
    out_specs=[
        pl.BlockSpec((BR6, FF), lambda i: (i, 0)),
        pl.BlockSpec((BR6, FF), lambda i: (i, 0)),
    ],
    out_shape=[
        jax.ShapeDtypeStruct((NN, FF), jnp.float32),
        jax.ShapeDtypeStruct((NN, FF), jnp.float32),
    ],
)


def _decoder_body(zr_ref, za_ref, out_ref):
    out_ref[...] = lax.dot_general(
        zr_ref[...], za_ref[...],
        dimension_numbers=(((1,), (1,)), ((), ())),
        preferred_element_type=jnp.float32,
    )


_decoder_call = pl.pallas_call(
    _decoder_body,
    grid=(NN // BRB,),
    in_specs=[
        pl.BlockSpec((BRB, FF), lambda i: (i, 0)),
        pl.BlockSpec((NN, FF), lambda i: (0, 0)),
    ],
    out_specs=pl.BlockSpec((BRB, NN), lambda i: (i, 0)),
    out_shape=jax.ShapeDtypeStruct((NN, NN), jnp.float32),
)


def kernel(x, edge_index, W1, b1, W2, b2):
    E = edge_index.shape[1]
    epw = -(-E // NW)
    K = -(-epw // CH)
    K += K % 2
    EP = NW * K * CH
    pad = EP - E

    src = edge_index[0]
    dst = edge_index[1]
    padidx = NN + (jnp.arange(pad, dtype=jnp.int32) % (NP - NN))
    src_p = jnp.concatenate([src, padidx]).reshape(NW, K, CH)
    dst_p = jnp.concatenate([dst, padidx]).reshape(NW, K, CH)
    x_pad = jnp.pad(x, ((0, NP - NN), (0, 0)))

    deg_call = _make_deg_kernel(K)
    prop_call = _make_prop_kernel(K)

    degp = deg_call(dst_p)
    degp_t = degp.T
    xw1 = _mm1_call(x_pad, W1)
    y1 = _scale1_call(degp_t, xw1)
    acc1 = prop_call(y1, src_p, dst_p)
    y2, hw2 = _l2_call(acc1, degp_t, xw1, b1.reshape(1, FF), W2)
    acc2 = prop_call(y2, src_p, dst_p)
    z, logp = _zlogp_call(acc2, degp_t, hw2, b2.reshape(1, FF))
    x_product = _decoder_call(z, z)
    return (logp, x_product)
